# baseline (speedup 1.0000x reference)
import functools

import jax
import jax.numpy as jnp
from jax import lax
from jax.experimental import pallas as pl
from jax.experimental.pallas import tpu as pltpu
from jax.experimental.pallas import tpu_sc as plsc

_N = 40000
_C = 128
_B = 4
_HF, _WF = 48, 160
_HI, _WI = 384, 1280
_STRIDE = 8
_MAX_BEV = 76.37
_HID = 64
_HW = _HF * _WF

_NPAD = 40960
_T = 2048
_GRID = _NPAD // _T
_NW = 16
_PER_W = _NPAD // _NW
_CH = 128
_NCH = _PER_W // _CH


def _stage1_body(xf, idf, tr_ref, geo_ref, fgw1, fgb1, fgw2r, fgb2,
                 rtw1a, rtw1b, rtb1, rtw2t, rtb2,
                 idx_out, w_out, aux_out):
    f32 = jnp.float32
    sx = idf[:, 3:4]
    sy = idf[:, 2:3]
    sz = idf[:, 1:2]
    bidf = idf[:, 0:1]

    sp3 = jnp.concatenate([sx, sy, sz], axis=1)
    cen = sp3 * geo_ref[0:1, :] + geo_ref[1:2, :] + geo_ref[2:3, :]
    cx = cen[:, 0:1]
    cy = cen[:, 1:2]
    cz = cen[:, 2:3]
    rng = jnp.sqrt(cx * cx + cy * cy)
    rn = jnp.clip(rng / _MAX_BEV, 0.0, 1.0)

    p0 = jnp.zeros_like(sx)
    p1 = jnp.zeros_like(sx)
    dep = jnp.zeros_like(sx)
    for b in range(_B):
        mb = (bidf == float(b)).astype(f32)

        def t(i, j, _b=b):
            return tr_ref[_b:_b + 1, 4 * i + j:4 * i + j + 1]

        q0 = t(0, 0) * cx + t(0, 1) * cy + t(0, 2) * cz + t(0, 3)
        q1 = t(1, 0) * cx + t(1, 1) * cy + t(1, 2) * cz + t(1, 3)
        q2 = t(2, 0) * cx + t(2, 1) * cy + t(2, 2) * cz + t(2, 3)
        p0 = p0 + mb * q0
        p1 = p1 + mb * q1
        dep = dep + mb * q2
    safe = jnp.maximum(dep, 1e-5)
    u_img = p0 / safe
    v_img = p1 / safe
    u_feat = u_img * (float(_WF) / float(_WI))
    v_feat = v_img * (float(_HF) / float(_HI))
    u_norm = 2.0 * (u_feat / float(_WF - 1)) - 1.0
    v_norm = 2.0 * (v_feat / float(_HF - 1)) - 1.0
    validf = ((dep > 1e-5) & (jnp.abs(u_norm) <= 1.0)
              & (jnp.abs(v_norm) <= 1.0)).astype(f32)

    gx = (u_norm + 1.0) * 0.5 * float(_WF - 1)
    gy = (v_norm + 1.0) * 0.5 * float(_HF - 1)
    x0 = jnp.floor(gx)
    y0 = jnp.floor(gy)
    x1 = x0 + 1.0
    y1 = y0 + 1.0
    wx1 = gx - x0
    wx0 = 1.0 - wx1
    wy1 = gy - y0
    wy0 = 1.0 - wy1
    idx_cols = []
    w_cols = []
    lane16 = jnp.ones((1, 16), f32)
    for xc, wx in ((x0, wx0), (x1, wx1)):
        for yc, wy in ((y0, wy0), (y1, wy1)):
            inb = ((xc >= 0.0) & (xc <= float(_WF - 1))
                   & (yc >= 0.0) & (yc <= float(_HF - 1))).astype(f32)
            xi = jnp.clip(xc, 0.0, float(_WF - 1))
            yi = jnp.clip(yc, 0.0, float(_HF - 1))
            fidx = bidf * float(_HW) + yi * float(_WF) + xi
            idx_cols.append(fidx.astype(jnp.int32))
            w_cols.append((wx * wy * inb) * lane16)
    idx_out[...] = jnp.concatenate(idx_cols, axis=1)
    w_out[...] = jnp.concatenate(w_cols, axis=1)

    x = xf[...]
    h1 = jnp.maximum(jnp.dot(x, fgw1[...], preferred_element_type=f32)
                     + fgb1[...], 0.0)
    pfg = jax.nn.sigmoid(jnp.sum(h1 * fgw2r[...], axis=1, keepdims=True)
                         + fgb2[...])

    h2 = (jnp.dot(x, rtw1a[...], preferred_element_type=f32)
          + rn * rtw1b[0:1, :] + pfg * rtw1b[1:2, :]
          + validf * rtw1b[2:3, :] + rtb1[...])
    h2 = jnp.maximum(h2, 0.0)
    l0 = jnp.sum(h2 * rtw2t[0:1, :], axis=1, keepdims=True) + rtb2[0:1, 0:1]
    l1 = jnp.sum(h2 * rtw2t[1:2, :], axis=1, keepdims=True) + rtb2[0:1, 1:2]
    l2 = jnp.sum(h2 * rtw2t[2:3, :], axis=1, keepdims=True) + rtb2[0:1, 2:3]
    m = jnp.maximum(jnp.maximum(l0, l1), l2)
    e0 = jnp.exp(l0 - m)
    e1 = jnp.exp(l1 - m)
    e2 = jnp.exp(l2 - m)
    s = e0 + e1 + e2
    z = jnp.zeros_like(pfg)
    aux_out[...] = jnp.concatenate(
        [e0 / s, e1 / s, e2 / s, pfg, rn, validf, z, z], axis=1)


def _sc_body(table, i0, i1, i2, i3, wexp, out,
             iv0, iv1, iv2, iv3, wvx,
             r0, r1, r2, r3, ov, sem):
    nc = 1
    wid = lax.axis_index("s") * nc + lax.axis_index("c")

    def chunk(c, carry):
        base = wid * _PER_W + c * _CH
        for src, dst in ((i0, iv0), (i1, iv1), (i2, iv2), (i3, iv3)):
            pltpu.sync_copy(src.at[pl.ds(base, _CH)], dst)
        pltpu.sync_copy(wexp.at[pl.ds(base, _CH)], wvx)
        cps = [pltpu.async_copy(table.at[iv], rr, sem)
               for iv, rr in ((iv0, r0), (iv1, r1), (iv2, r2), (iv3, r3))]
        for cp in cps:
            cp.wait()

        @plsc.parallel_loop(0, _CH, unroll=4)
        def _tok(i):
            a0 = wvx[i, pl.ds(0, 16)]
            a1 = wvx[i, pl.ds(16, 16)]
            a2 = wvx[i, pl.ds(32, 16)]
            a3 = wvx[i, pl.ds(48, 16)]
            for s in range(_C // 16):
                sl = pl.ds(s * 16, 16)
                acc = (a0 * r0[i, sl] + a1 * r1[i, sl]
                       + a2 * r2[i, sl] + a3 * r3[i, sl])
                ov[i, sl] = acc
        pltpu.sync_copy(ov, out.at[pl.ds(base, _CH)])
        return carry

    lax.fori_loop(0, _NCH, chunk, 0)


def _stage3_body(x_ref, s_ref, aux_ref, w1a, w1b, b1, w2, b2, g_ref, bl_ref,
                 out_ref, q_ref):
    f32 = jnp.float32
    x = x_ref[...]
    smp = s_ref[...]
    vf = aux_ref[:, 5:6]
    rw1 = aux_ref[:, 1:2]
    h = jnp.maximum(jnp.dot(x, w1a[...], preferred_element_type=f32)
                    + jnp.dot(smp, w1b[...], preferred_element_type=f32)
                    + b1[...], 0.0)
    dl = (jnp.dot(h, w2[...], preferred_element_type=f32) + b2[...]) * vf
    delta = rw1 * dl
    nd = jnp.sqrt(jnp.sum(delta * delta, axis=1, keepdims=True))
    nx = jnp.sqrt(jnp.sum(x * x, axis=1, keepdims=True))
    fg = jnp.clip(1.0 - jnp.exp(-(nd / (nx + 1e-6))), 0.0, 1.0)
    y = x + delta
    mu = jnp.mean(y, axis=1, keepdims=True)
    var = jnp.mean((y - mu) ** 2, axis=1, keepdims=True)
    out_ref[...] = (y - mu) / jnp.sqrt(var + 1e-5) * g_ref[...] + bl_ref[...]
    z = jnp.zeros_like(fg)
    q = jnp.concatenate(
        [aux_ref[:, 0:6], z, fg, z, z, z, z, z, z, z, z], axis=1)
    q_ref[...] = jnp.clip(q, 0.0, 1.0)


def _full_spec(r, c):
    return pl.BlockSpec((r, c), lambda i: (0, 0))


def _tok_spec(c):
    return pl.BlockSpec((_T, c), lambda i: (i, 0))


def kernel(features, indices, voxel_size, point_cloud_range,
           trans_lidar_to_img, images, img_feats,
           fg_w1, fg_b1, fg_w2, fg_b2, rt_w1, rt_b1, rt_w2, rt_b2,
           le_w1, le_b1, le_w2, le_b2, ln_g, ln_b):
    f32 = jnp.float32
    xp = jnp.pad(features, ((0, _NPAD - _N), (0, 0)))
    idf = jnp.pad(indices.astype(f32), ((0, _NPAD - _N), (0, 0)))

    cvs = voxel_size * float(_STRIDE)
    geo = jnp.stack([cvs, point_cloud_range[:3], cvs * 0.5])
    tr_flat = trans_lidar_to_img.reshape(_B, 16)

    idx_i32, w_f32, aux = pl.pallas_call(
        _stage1_body,
        grid=(_GRID,),
        in_specs=[
            _tok_spec(_C), _tok_spec(4),
            _full_spec(_B, 16), _full_spec(3, 3),
            _full_spec(_C, _HID), _full_spec(1, _HID),
            _full_spec(1, _HID), _full_spec(1, 1),
            _full_spec(_C, _HID), _full_spec(3, _HID),
            _full_spec(1, _HID), _full_spec(3, _HID), _full_spec(1, 3),
        ],
        out_specs=[_tok_spec(4), _tok_spec(64), _tok_spec(8)],
        out_shape=[
            jax.ShapeDtypeStruct((_NPAD, 4), jnp.int32),
            jax.ShapeDtypeStruct((_NPAD, 64), f32),
            jax.ShapeDtypeStruct((_NPAD, 8), f32),
        ],
    )(xp, idf, tr_flat, geo, fg_w1, fg_b1.reshape(1, _HID),
      fg_w2.reshape(1, _HID), fg_b2.reshape(1, 1),
      rt_w1[:_C], rt_w1[_C:], rt_b1.reshape(1, _HID),
      rt_w2.T, rt_b2.reshape(1, 3))

    table = img_feats.transpose(0, 2, 3, 1).reshape(_B * _HW, _C)
    i_cols = [idx_i32[:, k] for k in range(4)]

    sc_call = functools.partial(
        pl.kernel,
        out_type=jax.ShapeDtypeStruct((_NPAD, _C), f32),
        mesh=plsc.VectorSubcoreMesh(core_axis_name="c", subcore_axis_name="s", num_cores=1),
        scratch_types=[
            pltpu.VMEM((_CH,), jnp.int32),
            pltpu.VMEM((_CH,), jnp.int32),
            pltpu.VMEM((_CH,), jnp.int32),
            pltpu.VMEM((_CH,), jnp.int32),
            pltpu.VMEM((_CH, 64), f32),
            pltpu.VMEM((_CH, _C), f32),
            pltpu.VMEM((_CH, _C), f32),
            pltpu.VMEM((_CH, _C), f32),
            pltpu.VMEM((_CH, _C), f32),
            pltpu.VMEM((_CH, _C), f32),
            pltpu.SemaphoreType.DMA,
        ],
    )(_sc_body)
    sampled = sc_call(table, *i_cols, w_f32)

    out_p, q_p = pl.pallas_call(
        _stage3_body,
        grid=(_GRID,),
        in_specs=[
            _tok_spec(_C), _tok_spec(_C), _tok_spec(8),
            _full_spec(_C, _C), _full_spec(_C, _C), _full_spec(1, _C),
            _full_spec(_C, _C), _full_spec(1, _C),
            _full_spec(1, _C), _full_spec(1, _C),
        ],
        out_specs=[_tok_spec(_C), _tok_spec(16)],
        out_shape=[
            jax.ShapeDtypeStruct((_NPAD, _C), f32),
            jax.ShapeDtypeStruct((_NPAD, 16), f32),
        ],
    )(xp, sampled, aux, le_w1[:_C], le_w1[_C:], le_b1.reshape(1, _C),
      le_w2, le_b2.reshape(1, _C), ln_g.reshape(1, _C), ln_b.reshape(1, _C))

    return (out_p[:_N], q_p[:_N, :11])


# speedup vs baseline: 1.0607x; 1.0607x over previous
import functools

import jax
import jax.numpy as jnp
from jax import lax
from jax.experimental import pallas as pl
from jax.experimental.pallas import tpu as pltpu
from jax.experimental.pallas import tpu_sc as plsc

_N = 40000
_C = 128
_B = 4
_HF, _WF = 48, 160
_HI, _WI = 384, 1280
_STRIDE = 8
_MAX_BEV = 76.37
_HID = 64
_HW = _HF * _WF

_NPAD = 40960
_T = 2048
_GRID = _NPAD // _T
_NW = 32
_PER_W = _NPAD // _NW
_CH = 64
_NCH = _PER_W // _CH


def _stage1_body(xf, idf, tr_ref, geo_ref, fgw1, fgb1, fgw2r, fgb2,
                 rtw1a, rtw1b, rtb1, rtw2t, rtb2,
                 idx_out, w_out, aux_out):
    f32 = jnp.float32
    sx = idf[:, 3:4]
    sy = idf[:, 2:3]
    sz = idf[:, 1:2]
    bidf = idf[:, 0:1]

    sp3 = jnp.concatenate([sx, sy, sz], axis=1)
    cen = sp3 * geo_ref[0:1, :] + geo_ref[1:2, :] + geo_ref[2:3, :]
    cx = cen[:, 0:1]
    cy = cen[:, 1:2]
    cz = cen[:, 2:3]
    rng = jnp.sqrt(cx * cx + cy * cy)
    rn = jnp.clip(rng / _MAX_BEV, 0.0, 1.0)

    p0 = jnp.zeros_like(sx)
    p1 = jnp.zeros_like(sx)
    dep = jnp.zeros_like(sx)
    for b in range(_B):
        mb = (bidf == float(b)).astype(f32)

        def t(i, j, _b=b):
            return tr_ref[_b:_b + 1, 4 * i + j:4 * i + j + 1]

        q0 = t(0, 0) * cx + t(0, 1) * cy + t(0, 2) * cz + t(0, 3)
        q1 = t(1, 0) * cx + t(1, 1) * cy + t(1, 2) * cz + t(1, 3)
        q2 = t(2, 0) * cx + t(2, 1) * cy + t(2, 2) * cz + t(2, 3)
        p0 = p0 + mb * q0
        p1 = p1 + mb * q1
        dep = dep + mb * q2
    safe = jnp.maximum(dep, 1e-5)
    u_img = p0 / safe
    v_img = p1 / safe
    u_feat = u_img * (float(_WF) / float(_WI))
    v_feat = v_img * (float(_HF) / float(_HI))
    u_norm = 2.0 * (u_feat / float(_WF - 1)) - 1.0
    v_norm = 2.0 * (v_feat / float(_HF - 1)) - 1.0
    validf = ((dep > 1e-5) & (jnp.abs(u_norm) <= 1.0)
              & (jnp.abs(v_norm) <= 1.0)).astype(f32)

    gx = (u_norm + 1.0) * 0.5 * float(_WF - 1)
    gy = (v_norm + 1.0) * 0.5 * float(_HF - 1)
    x0 = jnp.floor(gx)
    y0 = jnp.floor(gy)
    x1 = x0 + 1.0
    y1 = y0 + 1.0
    wx1 = gx - x0
    wx0 = 1.0 - wx1
    wy1 = gy - y0
    wy0 = 1.0 - wy1
    idx_cols = []
    w_cols = []
    lane16 = jnp.ones((1, 16), f32)
    for xc, wx in ((x0, wx0), (x1, wx1)):
        for yc, wy in ((y0, wy0), (y1, wy1)):
            inb = ((xc >= 0.0) & (xc <= float(_WF - 1))
                   & (yc >= 0.0) & (yc <= float(_HF - 1))).astype(f32)
            xi = jnp.clip(xc, 0.0, float(_WF - 1))
            yi = jnp.clip(yc, 0.0, float(_HF - 1))
            fidx = bidf * float(_HW) + yi * float(_WF) + xi
            idx_cols.append(fidx.astype(jnp.int32))
            w_cols.append((wx * wy * inb) * lane16)
    idx_out[...] = jnp.concatenate(idx_cols, axis=1)
    w_out[...] = jnp.concatenate(w_cols, axis=1)

    x = xf[...]
    h1 = jnp.maximum(jnp.dot(x, fgw1[...], preferred_element_type=f32)
                     + fgb1[...], 0.0)
    pfg = jax.nn.sigmoid(jnp.sum(h1 * fgw2r[...], axis=1, keepdims=True)
                         + fgb2[...])

    h2 = (jnp.dot(x, rtw1a[...], preferred_element_type=f32)
          + rn * rtw1b[0:1, :] + pfg * rtw1b[1:2, :]
          + validf * rtw1b[2:3, :] + rtb1[...])
    h2 = jnp.maximum(h2, 0.0)
    l0 = jnp.sum(h2 * rtw2t[0:1, :], axis=1, keepdims=True) + rtb2[0:1, 0:1]
    l1 = jnp.sum(h2 * rtw2t[1:2, :], axis=1, keepdims=True) + rtb2[0:1, 1:2]
    l2 = jnp.sum(h2 * rtw2t[2:3, :], axis=1, keepdims=True) + rtb2[0:1, 2:3]
    m = jnp.maximum(jnp.maximum(l0, l1), l2)
    e0 = jnp.exp(l0 - m)
    e1 = jnp.exp(l1 - m)
    e2 = jnp.exp(l2 - m)
    s = e0 + e1 + e2
    z = jnp.zeros_like(pfg)
    aux_out[...] = jnp.concatenate(
        [e0 / s, e1 / s, e2 / s, pfg, rn, validf, z, z], axis=1)


def _sc_body(table, i0, i1, i2, i3, whb, out,
             iv0, iv1, iv2, iv3, wv0, wv1,
             r00, r01, r02, r03, r10, r11, r12, r13,
             ov0, ov1, sg0, sg1, sw0, sw1, so0, so1):
    nc = 2
    wid = lax.axis_index("s") * nc + lax.axis_index("c")
    ivs = (iv0, iv1, iv2, iv3)
    rows = ((r00, r01, r02, r03), (r10, r11, r12, r13))
    wvs = (wv0, wv1)
    ovs = (ov0, ov1)
    sgs = (sg0, sg1)
    sws = (sw0, sw1)
    sos = (so0, so1)

    for src, dst in ((i0, iv0), (i1, iv1), (i2, iv2), (i3, iv3)):
        pltpu.sync_copy(src.at[wid], dst)

    def issue(c, b):
        pltpu.async_copy(whb.at[wid, c], wvs[b], sws[b])
        for k in range(4):
            pltpu.async_copy(table.at[ivs[k].at[c]], rows[b][k], sgs[b])

    def drain(b):
        pltpu.make_async_copy(whb.at[wid, 0], wvs[b], sws[b]).wait()
        for k in range(4):
            pltpu.make_async_copy(
                table.at[ivs[k].at[0]], rows[b][k], sgs[b]).wait()

    def compute(b):
        wv = wvs[b]
        r0, r1, r2, r3 = rows[b]
        ov = ovs[b]

        @plsc.parallel_loop(0, _CH, unroll=4)
        def _tok(i):
            a0 = wv[i, pl.ds(0, 16)]
            a1 = wv[i, pl.ds(16, 16)]
            a2 = wv[i, pl.ds(32, 16)]
            a3 = wv[i, pl.ds(48, 16)]
            for s in range(_C // 16):
                sl = pl.ds(s * 16, 16)
                acc = (a0 * r0[i, sl] + a1 * r1[i, sl]
                       + a2 * r2[i, sl] + a3 * r3[i, sl])
                ov[i, sl] = acc

    issue(0, 0)
    issue(1, 1)

    def step(g, carry):
        for b in range(2):
            c = g * 2 + b
            drain(b)

            @pl.when(c >= 2)
            def _wait_flush(_b=b):
                pltpu.make_async_copy(
                    ovs[_b], out.at[pl.ds(0, _CH)], sos[_b]).wait()

            compute(b)
            base = wid * _PER_W + c * _CH
            pltpu.async_copy(ovs[b], out.at[pl.ds(base, _CH)], sos[b])

            @pl.when(c + 2 < _NCH)
            def _issue_next(_b=b, _c=c):
                issue(_c + 2, _b)
        return carry

    lax.fori_loop(0, _NCH // 2, step, 0)
    for b in range(2):
        pltpu.make_async_copy(ovs[b], out.at[pl.ds(0, _CH)], sos[b]).wait()


def _stage3_body(x_ref, s_ref, aux_ref, w1a, w1b, b1, w2, b2, g_ref, bl_ref,
                 out_ref, q_ref):
    f32 = jnp.float32
    x = x_ref[...]
    smp = s_ref[...]
    vf = aux_ref[:, 5:6]
    rw1 = aux_ref[:, 1:2]
    h = jnp.maximum(jnp.dot(x, w1a[...], preferred_element_type=f32)
                    + jnp.dot(smp, w1b[...], preferred_element_type=f32)
                    + b1[...], 0.0)
    dl = (jnp.dot(h, w2[...], preferred_element_type=f32) + b2[...]) * vf
    delta = rw1 * dl
    nd = jnp.sqrt(jnp.sum(delta * delta, axis=1, keepdims=True))
    nx = jnp.sqrt(jnp.sum(x * x, axis=1, keepdims=True))
    fg = jnp.clip(1.0 - jnp.exp(-(nd / (nx + 1e-6))), 0.0, 1.0)
    y = x + delta
    mu = jnp.mean(y, axis=1, keepdims=True)
    var = jnp.mean((y - mu) ** 2, axis=1, keepdims=True)
    out_ref[...] = (y - mu) / jnp.sqrt(var + 1e-5) * g_ref[...] + bl_ref[...]
    z = jnp.zeros_like(fg)
    q = jnp.concatenate(
        [aux_ref[:, 0:6], z, fg, z, z, z, z, z, z, z, z], axis=1)
    q_ref[...] = jnp.clip(q, 0.0, 1.0)


def _full_spec(r, c):
    return pl.BlockSpec((r, c), lambda i: (0, 0))


def _tok_spec(c):
    return pl.BlockSpec((_T, c), lambda i: (i, 0))


def kernel(features, indices, voxel_size, point_cloud_range,
           trans_lidar_to_img, images, img_feats,
           fg_w1, fg_b1, fg_w2, fg_b2, rt_w1, rt_b1, rt_w2, rt_b2,
           le_w1, le_b1, le_w2, le_b2, ln_g, ln_b):
    f32 = jnp.float32
    xp = jnp.pad(features, ((0, _NPAD - _N), (0, 0)))
    idf = jnp.pad(indices.astype(f32), ((0, _NPAD - _N), (0, 0)))

    cvs = voxel_size * float(_STRIDE)
    geo = jnp.stack([cvs, point_cloud_range[:3], cvs * 0.5])
    tr_flat = trans_lidar_to_img.reshape(_B, 16)

    idx_i32, w_f32, aux = pl.pallas_call(
        _stage1_body,
        grid=(_GRID,),
        in_specs=[
            _tok_spec(_C), _tok_spec(4),
            _full_spec(_B, 16), _full_spec(3, 3),
            _full_spec(_C, _HID), _full_spec(1, _HID),
            _full_spec(1, _HID), _full_spec(1, 1),
            _full_spec(_C, _HID), _full_spec(3, _HID),
            _full_spec(1, _HID), _full_spec(3, _HID), _full_spec(1, 3),
        ],
        out_specs=[_tok_spec(4), _tok_spec(64), _tok_spec(8)],
        out_shape=[
            jax.ShapeDtypeStruct((_NPAD, 4), jnp.int32),
            jax.ShapeDtypeStruct((_NPAD, 64), f32),
            jax.ShapeDtypeStruct((_NPAD, 8), f32),
        ],
    )(xp, idf, tr_flat, geo, fg_w1, fg_b1.reshape(1, _HID),
      fg_w2.reshape(1, _HID), fg_b2.reshape(1, 1),
      rt_w1[:_C], rt_w1[_C:], rt_b1.reshape(1, _HID),
      rt_w2.T, rt_b2.reshape(1, 3))

    table = img_feats.transpose(0, 2, 3, 1).reshape(_B * _HW, _C)
    i_arrs = [idx_i32[:, k].reshape(_NW, _NCH, _CH) for k in range(4)]
    w_arr = w_f32.reshape(_NW, _NCH, _CH, 64)

    rows_t = [pltpu.VMEM((_CH, _C), f32) for _ in range(8)]
    sc_call = functools.partial(
        pl.kernel,
        out_type=jax.ShapeDtypeStruct((_NPAD, _C), f32),
        mesh=plsc.VectorSubcoreMesh(core_axis_name="c", subcore_axis_name="s"),
        scratch_types=[
            pltpu.VMEM((_NCH, _CH), jnp.int32),
            pltpu.VMEM((_NCH, _CH), jnp.int32),
            pltpu.VMEM((_NCH, _CH), jnp.int32),
            pltpu.VMEM((_NCH, _CH), jnp.int32),
            pltpu.VMEM((_CH, 64), f32),
            pltpu.VMEM((_CH, 64), f32),
            *rows_t,
            pltpu.VMEM((_CH, _C), f32),
            pltpu.VMEM((_CH, _C), f32),
            pltpu.SemaphoreType.DMA,
            pltpu.SemaphoreType.DMA,
            pltpu.SemaphoreType.DMA,
            pltpu.SemaphoreType.DMA,
            pltpu.SemaphoreType.DMA,
            pltpu.SemaphoreType.DMA,
        ],
    )(_sc_body)
    sampled = sc_call(table, *i_arrs, w_arr)

    out_p, q_p = pl.pallas_call(
        _stage3_body,
        grid=(_GRID,),
        in_specs=[
            _tok_spec(_C), _tok_spec(_C), _tok_spec(8),
            _full_spec(_C, _C), _full_spec(_C, _C), _full_spec(1, _C),
            _full_spec(_C, _C), _full_spec(1, _C),
            _full_spec(1, _C), _full_spec(1, _C),
        ],
        out_specs=[_tok_spec(_C), _tok_spec(16)],
        out_shape=[
            jax.ShapeDtypeStruct((_NPAD, _C), f32),
            jax.ShapeDtypeStruct((_NPAD, 16), f32),
        ],
    )(xp, sampled, aux, le_w1[:_C], le_w1[_C:], le_b1.reshape(1, _C),
      le_w2, le_b2.reshape(1, _C), ln_g.reshape(1, _C), ln_b.reshape(1, _C))

    return (out_p[:_N], q_p[:_N, :11])


# speedup vs baseline: 1.5873x; 1.4965x over previous
import functools

import numpy as np

import jax
import jax.numpy as jnp
from jax import lax
from jax.experimental import pallas as pl
from jax.experimental.pallas import tpu as pltpu
from jax.experimental.pallas import tpu_sc as plsc

_N = 40000
_C = 128
_B = 4
_HF, _WF = 48, 160
_HI, _WI = 384, 1280
_STRIDE = 8
_MAX_BEV = 76.37
_HID = 64
_HW = _HF * _WF

_NPAD = 40960
_T = 2048
_GRID = _NPAD // _T
_NW = 32
_PER_W = _NPAD // _NW
_CH = 64
_NCH = _PER_W // _CH
_IVR = _PER_W // 128


def _stage1_body(xf, idf, tr_ref, geo_ref, fgw1, fgb1, fgw2r, fgb2,
                 rtw1a, rtw1b, rtb1, rtw2t, rtb2,
                 idx_out, w_out, aux_out):
    f32 = jnp.float32
    sx = idf[:, 3:4]
    sy = idf[:, 2:3]
    sz = idf[:, 1:2]
    bidf = idf[:, 0:1]

    sp3 = jnp.concatenate([sx, sy, sz], axis=1)
    cen = sp3 * geo_ref[0:1, :] + geo_ref[1:2, :] + geo_ref[2:3, :]
    cx = cen[:, 0:1]
    cy = cen[:, 1:2]
    cz = cen[:, 2:3]
    rng = jnp.sqrt(cx * cx + cy * cy)
    rn = jnp.clip(rng / _MAX_BEV, 0.0, 1.0)

    p0 = jnp.zeros_like(sx)
    p1 = jnp.zeros_like(sx)
    dep = jnp.zeros_like(sx)
    for b in range(_B):
        mb = (bidf == float(b)).astype(f32)

        def t(i, j, _b=b):
            return tr_ref[_b:_b + 1, 4 * i + j:4 * i + j + 1]

        q0 = t(0, 0) * cx + t(0, 1) * cy + t(0, 2) * cz + t(0, 3)
        q1 = t(1, 0) * cx + t(1, 1) * cy + t(1, 2) * cz + t(1, 3)
        q2 = t(2, 0) * cx + t(2, 1) * cy + t(2, 2) * cz + t(2, 3)
        p0 = p0 + mb * q0
        p1 = p1 + mb * q1
        dep = dep + mb * q2
    safe = jnp.maximum(dep, 1e-5)
    u_img = p0 / safe
    v_img = p1 / safe
    u_feat = u_img * (float(_WF) / float(_WI))
    v_feat = v_img * (float(_HF) / float(_HI))
    u_norm = 2.0 * (u_feat / float(_WF - 1)) - 1.0
    v_norm = 2.0 * (v_feat / float(_HF - 1)) - 1.0
    validf = ((dep > 1e-5) & (jnp.abs(u_norm) <= 1.0)
              & (jnp.abs(v_norm) <= 1.0)).astype(f32)

    gx = (u_norm + 1.0) * 0.5 * float(_WF - 1)
    gy = (v_norm + 1.0) * 0.5 * float(_HF - 1)
    x0 = jnp.floor(gx)
    y0 = jnp.floor(gy)
    x1 = x0 + 1.0
    y1 = y0 + 1.0
    wx1 = gx - x0
    wx0 = 1.0 - wx1
    wy1 = gy - y0
    wy0 = 1.0 - wy1
    px = jnp.clip(x0, 0.0, float(_WF - 1))
    py = jnp.clip(y0, 0.0, float(_HF - 1))
    qidx = (bidf * float(_HW) + py * float(_WF) + px).astype(jnp.int32)
    zero1 = jnp.zeros_like(gx)

    def contrib(pc, c0, c1, w0c, w1c, hi):
        in0 = (c0 >= 0.0) & (c0 <= hi)
        in1 = (c1 >= 0.0) & (c1 <= hi)
        return (jnp.where((pc == c0) & in0, w0c, zero1)
                + jnp.where((pc == c1) & in1, w1c, zero1))

    w_cols = []
    lane16 = jnp.ones((1, 16), f32)
    for dx in (0.0, 1.0):
        wxc = contrib(px + dx, x0, x1, wx0, wx1, float(_WF - 1))
        for dy in (0.0, 1.0):
            wyc = contrib(py + dy, y0, y1, wy0, wy1, float(_HF - 1))
            w_cols.append((wxc * wyc) * lane16)
    idx_out[...] = jnp.concatenate([qidx, qidx, qidx, qidx], axis=1)
    w_out[...] = jnp.concatenate(w_cols, axis=1)

    x = xf[...]
    h1 = jnp.maximum(jnp.dot(x, fgw1[...], preferred_element_type=f32)
                     + fgb1[...], 0.0)
    pfg = jax.nn.sigmoid(jnp.sum(h1 * fgw2r[...], axis=1, keepdims=True)
                         + fgb2[...])

    h2 = (jnp.dot(x, rtw1a[...], preferred_element_type=f32)
          + rn * rtw1b[0:1, :] + pfg * rtw1b[1:2, :]
          + validf * rtw1b[2:3, :] + rtb1[...])
    h2 = jnp.maximum(h2, 0.0)
    l0 = jnp.sum(h2 * rtw2t[0:1, :], axis=1, keepdims=True) + rtb2[0:1, 0:1]
    l1 = jnp.sum(h2 * rtw2t[1:2, :], axis=1, keepdims=True) + rtb2[0:1, 1:2]
    l2 = jnp.sum(h2 * rtw2t[2:3, :], axis=1, keepdims=True) + rtb2[0:1, 2:3]
    m = jnp.maximum(jnp.maximum(l0, l1), l2)
    e0 = jnp.exp(l0 - m)
    e1 = jnp.exp(l1 - m)
    e2 = jnp.exp(l2 - m)
    s = e0 + e1 + e2
    z = jnp.zeros_like(pfg)
    aux_out[...] = jnp.concatenate(
        [e0 / s, e1 / s, e2 / s, pfg, rn, validf, z, z], axis=1)


def _sc_body(t4, i0, whb, out,
             iv, wv0, wv1, r0, r1, ov0, ov1,
             sg0, sg1, sw0, sw1, so0, so1):
    nc = 2
    wid = lax.axis_index("s") * nc + lax.axis_index("c")
    rows = (r0, r1)
    wvs = (wv0, wv1)
    ovs = (ov0, ov1)
    sgs = (sg0, sg1)
    sws = (sw0, sw1)
    sos = (so0, so1)

    pltpu.sync_copy(i0.at[wid], iv)

    def issue(c, b):
        pltpu.async_copy(whb.at[wid, c], wvs[b], sws[b])
        r_i = c // (128 // _CH)
        off = (c % (128 // _CH)) * _CH
        pltpu.async_copy(t4.at[iv.at[r_i, pl.ds(off, _CH)]], rows[b], sgs[b])

    def drain(b):
        pltpu.make_async_copy(whb.at[0, 0], wvs[b], sws[b]).wait()
        pltpu.make_async_copy(
            t4.at[iv.at[0, pl.ds(0, _CH)]], rows[b], sgs[b]).wait()

    def compute(b):
        wv = wvs[b]
        r = rows[b]
        ov = ovs[b]

        @plsc.parallel_loop(0, _CH, unroll=4)
        def _tok(i):
            a0 = wv[i, pl.ds(0, 16)]
            a1 = wv[i, pl.ds(16, 16)]
            a2 = wv[i, pl.ds(32, 16)]
            a3 = wv[i, pl.ds(48, 16)]
            for si in range(_C // 16):
                acc = (a0 * r[i, pl.ds(si * 16, 16)]
                       + a1 * r[i, pl.ds(_C + si * 16, 16)]
                       + a2 * r[i, pl.ds(2 * _C + si * 16, 16)]
                       + a3 * r[i, pl.ds(3 * _C + si * 16, 16)])
                ov[i, pl.ds(si * 16, 16)] = acc

    issue(0, 0)
    issue(1, 1)

    def step(g, carry):
        for b in range(2):
            c = g * 2 + b
            drain(b)

            @pl.when(c >= 2)
            def _wait_flush(_b=b):
                pltpu.make_async_copy(
                    ovs[_b], out.at[pl.ds(0, _CH)], sos[_b]).wait()

            compute(b)
            base = wid * _PER_W + c * _CH
            pltpu.async_copy(ovs[b], out.at[pl.ds(base, _CH)], sos[b])

            @pl.when(c + 2 < _NCH)
            def _issue_next(_b=b, _c=c):
                issue(_c + 2, _b)
        return carry

    lax.fori_loop(0, _NCH // 2, step, 0)
    for b in range(2):
        pltpu.make_async_copy(ovs[b], out.at[pl.ds(0, _CH)], sos[b]).wait()


def _stage3_body(x_ref, s_ref, aux_ref, w1a, w1b, b1, w2, b2, g_ref, bl_ref,
                 out_ref, q_ref):
    f32 = jnp.float32
    x = x_ref[...]
    smp = s_ref[...].astype(f32)
    vf = aux_ref[:, 5:6]
    rw1 = aux_ref[:, 1:2]
    h = jnp.maximum(jnp.dot(x, w1a[...], preferred_element_type=f32)
                    + jnp.dot(smp, w1b[...], preferred_element_type=f32)
                    + b1[...], 0.0)
    dl = (jnp.dot(h, w2[...], preferred_element_type=f32) + b2[...]) * vf
    delta = rw1 * dl
    nd = jnp.sqrt(jnp.sum(delta * delta, axis=1, keepdims=True))
    nx = jnp.sqrt(jnp.sum(x * x, axis=1, keepdims=True))
    fg = jnp.clip(1.0 - jnp.exp(-(nd / (nx + 1e-6))), 0.0, 1.0)
    y = x + delta
    mu = jnp.mean(y, axis=1, keepdims=True)
    var = jnp.mean((y - mu) ** 2, axis=1, keepdims=True)
    out_ref[...] = (y - mu) / jnp.sqrt(var + 1e-5) * g_ref[...] + bl_ref[...]
    z = jnp.zeros_like(fg)
    q = jnp.concatenate(
        [aux_ref[:, 0:6], z, fg, z, z, z, z, z, z, z, z], axis=1)
    q_ref[...] = jnp.clip(q, 0.0, 1.0)


def _full_spec(r, c):
    return pl.BlockSpec((r, c), lambda i: (0, 0))


def _tok_spec(c):
    return pl.BlockSpec((_T, c), lambda i: (i, 0))


def kernel(features, indices, voxel_size, point_cloud_range,
           trans_lidar_to_img, images, img_feats,
           fg_w1, fg_b1, fg_w2, fg_b2, rt_w1, rt_b1, rt_w2, rt_b2,
           le_w1, le_b1, le_w2, le_b2, ln_g, ln_b):
    f32 = jnp.float32
    xp = jnp.pad(features, ((0, _NPAD - _N), (0, 0)))
    idf = jnp.pad(indices.astype(f32), ((0, _NPAD - _N), (0, 0)))

    cvs = voxel_size * float(_STRIDE)
    geo = jnp.stack([cvs, point_cloud_range[:3], cvs * 0.5])
    tr_flat = trans_lidar_to_img.reshape(_B, 16)

    idx_i32, w_f32, aux = pl.pallas_call(
        _stage1_body,
        grid=(_GRID,),
        in_specs=[
            _tok_spec(_C), _tok_spec(4),
            _full_spec(_B, 16), _full_spec(3, 3),
            _full_spec(_C, _HID), _full_spec(1, _HID),
            _full_spec(1, _HID), _full_spec(1, 1),
            _full_spec(_C, _HID), _full_spec(3, _HID),
            _full_spec(1, _HID), _full_spec(3, _HID), _full_spec(1, 3),
        ],
        out_specs=[_tok_spec(4), _tok_spec(64), _tok_spec(8)],
        out_shape=[
            jax.ShapeDtypeStruct((_NPAD, 4), jnp.int32),
            jax.ShapeDtypeStruct((_NPAD, 64), f32),
            jax.ShapeDtypeStruct((_NPAD, 8), f32),
        ],
    )(xp, idf, tr_flat, geo, fg_w1, fg_b1.reshape(1, _HID),
      fg_w2.reshape(1, _HID), fg_b2.reshape(1, 1),
      rt_w1[:_C], rt_w1[_C:], rt_b1.reshape(1, _HID),
      rt_w2.T, rt_b2.reshape(1, 3))

    flat = img_feats.transpose(0, 2, 3, 1).reshape(_B * _HW, _C)
    t4 = jnp.concatenate(
        [flat,
         jnp.roll(flat, -_WF, axis=0),
         jnp.roll(flat, -1, axis=0),
         jnp.roll(flat, -(_WF + 1), axis=0)], axis=1)
    i_arr = idx_i32[:, 0].reshape(_NW, _IVR, 128)
    w_arr = w_f32.reshape(_NW, _NCH, _CH, 64)

    sc_call = functools.partial(
        pl.kernel,
        out_type=jax.ShapeDtypeStruct((_NPAD, _C), f32),
        mesh=plsc.VectorSubcoreMesh(core_axis_name="c", subcore_axis_name="s"),
        scratch_types=[
            pltpu.VMEM((_IVR, 128), jnp.int32),
            pltpu.VMEM((_CH, 64), f32),
            pltpu.VMEM((_CH, 64), f32),
            pltpu.VMEM((_CH, 4 * _C), f32),
            pltpu.VMEM((_CH, 4 * _C), f32),
            pltpu.VMEM((_CH, _C), f32),
            pltpu.VMEM((_CH, _C), f32),
            pltpu.SemaphoreType.DMA,
            pltpu.SemaphoreType.DMA,
            pltpu.SemaphoreType.DMA,
            pltpu.SemaphoreType.DMA,
            pltpu.SemaphoreType.DMA,
            pltpu.SemaphoreType.DMA,
        ],
    )(_sc_body)
    sampled = sc_call(t4, i_arr, w_arr)

    out_p, q_p = pl.pallas_call(
        _stage3_body,
        grid=(_GRID,),
        in_specs=[
            _tok_spec(_C), _tok_spec(_C), _tok_spec(8),
            _full_spec(_C, _C), _full_spec(_C, _C), _full_spec(1, _C),
            _full_spec(_C, _C), _full_spec(1, _C),
            _full_spec(1, _C), _full_spec(1, _C),
        ],
        out_specs=[_tok_spec(_C), _tok_spec(16)],
        out_shape=[
            jax.ShapeDtypeStruct((_NPAD, _C), f32),
            jax.ShapeDtypeStruct((_NPAD, 16), f32),
        ],
    )(xp, sampled, aux, le_w1[:_C], le_w1[_C:], le_b1.reshape(1, _C),
      le_w2, le_b2.reshape(1, _C), ln_g.reshape(1, _C), ln_b.reshape(1, _C))

    return (out_p[:_N], q_p[:_N, :11])
